# Initial kernel scaffold; baseline (speedup 1.0000x reference)
import jax
import jax.numpy as jnp
from jax.experimental import pallas as pl


def kernel(z, W):
    raise NotImplementedError("write your pallas kernel here")


# speedup vs baseline: 1.0297x; 1.0297x over previous
import functools

import jax
import jax.numpy as jnp
from jax import lax
from jax.experimental import pallas as pl
from jax.experimental.pallas import tpu as pltpu
from jax.experimental.pallas import tpu_sc as plsc

B, H, W_SP, D, K = 8, 32, 32, 32, 8192
N = B * H * W_SP
TK = 512
NUM_K = K // TK


def _argmin_body(z_ref, w_ref, idx_ref, loss_ref, zsq_ref, best_v, best_i):
    k = pl.program_id(0)
    zf = z_ref[...]

    @pl.when(k == 0)
    def _init_zsq():
        zsq_ref[...] = jnp.sum(zf * zf, axis=1, keepdims=True)

    wblk = w_ref[...]
    csq = jnp.sum(wblk * wblk, axis=0, keepdims=True)
    s = lax.dot_general(zf, wblk, (((1,), (0,)), ((), ())),
                        preferred_element_type=jnp.float32)
    d = (zsq_ref[...] + csq) - 2.0 * s
    m = jnp.min(d, axis=1, keepdims=True)
    lane = lax.broadcasted_iota(jnp.int32, d.shape, 1)
    a = jnp.min(jnp.where(d == m, lane, TK), axis=1, keepdims=True)
    gidx = k * TK + a

    @pl.when(k == 0)
    def _init_best():
        best_v[...] = m
        best_i[...] = gidx

    @pl.when(k > 0)
    def _update_best():
        better = m < best_v[...]
        best_v[...] = jnp.where(better, m, best_v[...])
        best_i[...] = jnp.where(better, gidx, best_i[...])

    @pl.when(k == NUM_K - 1)
    def _finalize():
        idx_ref[...] = best_i[...]
        loss_ref[...] = jnp.sum(best_v[...], keepdims=True).reshape(1, 1) / (N * D)


def _tc_argmin(zf, W):
    return pl.pallas_call(
        _argmin_body,
        grid=(NUM_K,),
        in_specs=[
            pl.BlockSpec((N, D), lambda k: (0, 0)),
            pl.BlockSpec((D, TK), lambda k: (0, k)),
        ],
        out_specs=[
            pl.BlockSpec((N, 1), lambda k: (0, 0)),
            pl.BlockSpec((1, 1), lambda k: (0, 0)),
        ],
        out_shape=[
            jax.ShapeDtypeStruct((N, 1), jnp.int32),
            jax.ShapeDtypeStruct((1, 1), jnp.float32),
        ],
        scratch_shapes=[
            pltpu.VMEM((N, 1), jnp.float32),
            pltpu.VMEM((N, 1), jnp.float32),
            pltpu.VMEM((N, 1), jnp.int32),
        ],
    )(zf, W)


def _make_sc_gather():
    info = plsc.get_sparse_core_info()
    nc, ns = info.num_cores, info.num_subcores
    nw = nc * ns
    b_per_w = N // nw
    chunks = b_per_w // 128
    mesh = plsc.VectorSubcoreMesh(core_axis_name="c", subcore_axis_name="s")

    @functools.partial(
        pl.kernel,
        mesh=mesh,
        compiler_params=pltpu.CompilerParams(use_tc_tiling_on_sc=False),
        out_type=jax.ShapeDtypeStruct((N, D), jnp.float32),
        scratch_types=[
            pltpu.VMEM((chunks, 128), jnp.int32),
            pltpu.VMEM((b_per_w, D), jnp.float32),
            pltpu.SemaphoreType.DMA,
        ],
    )
    def gather(table_hbm, idx_hbm, out_hbm, idx_v, rows_v, sem):
        wid = lax.axis_index("s") * nc + lax.axis_index("c")
        base = wid * b_per_w
        pltpu.sync_copy(idx_hbm.at[wid], idx_v)
        for j in range(chunks):
            pltpu.async_copy(table_hbm.at[idx_v.at[j]],
                             rows_v.at[pl.ds(j * 128, 128)], sem).wait()
        pltpu.sync_copy(rows_v, out_hbm.at[pl.ds(base, b_per_w)])

    return gather, nw, b_per_w, chunks


def kernel(z, W):
    zf = z.reshape(N, D)
    idx2, loss_arr = _tc_argmin(zf, W)
    gather, nw, b_per_w, chunks = _make_sc_gather()
    table = W.T
    idx3 = idx2.reshape(nw, chunks, 128)
    q = gather(table, idx3)
    quantized_st = q.reshape(B, H, W_SP, D).transpose(0, 3, 1, 2)
    loss = loss_arr[0, 0]
    return quantized_st, loss, loss


# speedup vs baseline: 1.7842x; 1.7328x over previous
import functools

import jax
import jax.numpy as jnp
from jax import lax
from jax.experimental import pallas as pl
from jax.experimental.pallas import tpu as pltpu
from jax.experimental.pallas import tpu_sc as plsc

B, H, W_SP, D, K = 8, 32, 32, 32, 8192
N = B * H * W_SP
TK = 512
NUM_K = K // TK


def _argmin_body(z_ref, w_ref, idx_ref, loss_ref, zsq_ref, acc_v, acc_i):
    k = pl.program_id(0)
    zf = z_ref[...]

    @pl.when(k == 0)
    def _init():
        zsq_ref[...] = jnp.sum(zf * zf, axis=1, keepdims=True)
        acc_v[...] = jnp.full((N, 128), jnp.inf, jnp.float32)
        acc_i[...] = jnp.zeros((N, 128), jnp.int32)

    wblk = w_ref[...]
    csq = jnp.sum(wblk * wblk, axis=0, keepdims=True)
    s = lax.dot_general(zf, wblk, (((1,), (0,)), ((), ())),
                        preferred_element_type=jnp.float32)
    zsq = zsq_ref[...]
    lane = lax.broadcasted_iota(jnp.int32, (1, 128), 1)
    av, ai = acc_v[...], acc_i[...]
    for j in range(TK // 128):
        d = (zsq + csq[:, j * 128:(j + 1) * 128]) - 2.0 * s[:, j * 128:(j + 1) * 128]
        gidx = lane + (k * TK + j * 128)
        better = d < av
        av = jnp.minimum(d, av)
        ai = jnp.where(better, gidx, ai)
    acc_v[...] = av
    acc_i[...] = ai

    @pl.when(k == NUM_K - 1)
    def _finalize():
        m = jnp.min(av, axis=1, keepdims=True)
        cand = jnp.where(av == m, ai, jnp.int32(2**30))
        idx_ref[...] = jnp.min(cand, axis=1, keepdims=True)
        loss_ref[...] = jnp.sum(m, keepdims=True).reshape(1, 1) / (N * D)


def _tc_argmin(zf, W):
    return pl.pallas_call(
        _argmin_body,
        grid=(NUM_K,),
        in_specs=[
            pl.BlockSpec((N, D), lambda k: (0, 0)),
            pl.BlockSpec((D, TK), lambda k: (0, k)),
        ],
        out_specs=[
            pl.BlockSpec((N, 1), lambda k: (0, 0)),
            pl.BlockSpec((1, 1), lambda k: (0, 0)),
        ],
        out_shape=[
            jax.ShapeDtypeStruct((N, 1), jnp.int32),
            jax.ShapeDtypeStruct((1, 1), jnp.float32),
        ],
        scratch_shapes=[
            pltpu.VMEM((N, 1), jnp.float32),
            pltpu.VMEM((N, 128), jnp.float32),
            pltpu.VMEM((N, 128), jnp.int32),
        ],
    )(zf, W)


def _make_sc_gather():
    info = plsc.get_sparse_core_info()
    nc, ns = info.num_cores, info.num_subcores
    nw = nc * ns
    b_per_w = N // nw
    chunks = b_per_w // 128
    mesh = plsc.VectorSubcoreMesh(core_axis_name="c", subcore_axis_name="s")

    @functools.partial(
        pl.kernel,
        mesh=mesh,
        compiler_params=pltpu.CompilerParams(use_tc_tiling_on_sc=False),
        out_type=jax.ShapeDtypeStruct((N, D), jnp.float32),
        scratch_types=[
            pltpu.VMEM((chunks, 128), jnp.int32),
            pltpu.VMEM((b_per_w, D), jnp.float32),
            pltpu.SemaphoreType.DMA,
        ],
    )
    def gather(table_hbm, idx_hbm, out_hbm, idx_v, rows_v, sem):
        wid = lax.axis_index("s") * nc + lax.axis_index("c")
        base = wid * b_per_w
        pltpu.sync_copy(idx_hbm.at[wid], idx_v)
        for j in range(chunks):
            pltpu.async_copy(table_hbm.at[idx_v.at[j]],
                             rows_v.at[pl.ds(j * 128, 128)], sem).wait()
        pltpu.sync_copy(rows_v, out_hbm.at[pl.ds(base, b_per_w)])

    return gather, nw, b_per_w, chunks


def kernel(z, W):
    zf = z.reshape(N, D)
    idx2, loss_arr = _tc_argmin(zf, W)
    gather, nw, b_per_w, chunks = _make_sc_gather()
    table = W.T
    idx3 = idx2.reshape(nw, chunks, 128)
    q = gather(table, idx3)
    quantized_st = q.reshape(B, H, W_SP, D).transpose(0, 3, 1, 2)
    loss = loss_arr[0, 0]
    return quantized_st, loss, loss


# speedup vs baseline: 1.8092x; 1.0140x over previous
import functools

import jax
import jax.numpy as jnp
from jax import lax
from jax.experimental import pallas as pl
from jax.experimental.pallas import tpu as pltpu
from jax.experimental.pallas import tpu_sc as plsc

B, H, W_SP, D, K = 8, 32, 32, 32, 8192
N = B * H * W_SP
TK = 512
NUM_K = K // TK


def _argmin_body(z_ref, w_ref, idx_ref, loss_ref, zsq_ref, acc_v, acc_i):
    k = pl.program_id(0)
    zf = z_ref[...]

    @pl.when(k == 0)
    def _init():
        zsq_ref[...] = jnp.sum(zf * zf, axis=1, keepdims=True)
        acc_v[...] = jnp.full((N, 128), jnp.inf, jnp.float32)
        acc_i[...] = jnp.zeros((N, 128), jnp.int32)

    wblk = w_ref[...]
    csq = jnp.sum(wblk * wblk, axis=0, keepdims=True)
    s = lax.dot_general(zf, wblk, (((1,), (0,)), ((), ())),
                        preferred_element_type=jnp.float32)
    zsq = zsq_ref[...]
    lane = lax.broadcasted_iota(jnp.int32, (1, 128), 1)
    pairs = []
    for j in range(TK // 128):
        d = (zsq + csq[:, j * 128:(j + 1) * 128]) - 2.0 * s[:, j * 128:(j + 1) * 128]
        gidx = jnp.broadcast_to(lane + (k * TK + j * 128), (N, 128))
        pairs.append((d, gidx))
    while len(pairs) > 1:
        nxt = []
        for a in range(0, len(pairs), 2):
            (dv0, di0), (dv1, di1) = pairs[a], pairs[a + 1]
            better = dv1 < dv0
            nxt.append((jnp.minimum(dv0, dv1), jnp.where(better, di1, di0)))
        pairs = nxt
    dloc, iloc = pairs[0]
    better = dloc < acc_v[...]
    acc_v[...] = jnp.minimum(dloc, acc_v[...])
    acc_i[...] = jnp.where(better, iloc, acc_i[...])

    @pl.when(k == NUM_K - 1)
    def _finalize():
        av, ai = acc_v[...], acc_i[...]
        m = jnp.min(av, axis=1, keepdims=True)
        cand = jnp.where(av == m, ai, jnp.int32(2**30))
        idx_ref[...] = jnp.min(cand, axis=1, keepdims=True)
        loss_ref[...] = jnp.sum(m, keepdims=True).reshape(1, 1) / (N * D)


def _tc_argmin(zf, W):
    return pl.pallas_call(
        _argmin_body,
        grid=(NUM_K,),
        in_specs=[
            pl.BlockSpec((N, D), lambda k: (0, 0)),
            pl.BlockSpec((D, TK), lambda k: (0, k)),
        ],
        out_specs=[
            pl.BlockSpec((N, 1), lambda k: (0, 0)),
            pl.BlockSpec((1, 1), lambda k: (0, 0)),
        ],
        out_shape=[
            jax.ShapeDtypeStruct((N, 1), jnp.int32),
            jax.ShapeDtypeStruct((1, 1), jnp.float32),
        ],
        scratch_shapes=[
            pltpu.VMEM((N, 1), jnp.float32),
            pltpu.VMEM((N, 128), jnp.float32),
            pltpu.VMEM((N, 128), jnp.int32),
        ],
    )(zf, W)


def _make_sc_gather():
    info = plsc.get_sparse_core_info()
    nc, ns = info.num_cores, info.num_subcores
    nw = nc * ns
    b_per_w = N // nw
    chunks = b_per_w // 128
    mesh = plsc.VectorSubcoreMesh(core_axis_name="c", subcore_axis_name="s")

    @functools.partial(
        pl.kernel,
        mesh=mesh,
        compiler_params=pltpu.CompilerParams(use_tc_tiling_on_sc=False),
        out_type=jax.ShapeDtypeStruct((N, D), jnp.float32),
        scratch_types=[
            pltpu.VMEM((chunks, 128), jnp.int32),
            pltpu.VMEM((b_per_w, D), jnp.float32),
            pltpu.SemaphoreType.DMA,
        ],
    )
    def gather(table_hbm, idx_hbm, out_hbm, idx_v, rows_v, sem):
        wid = lax.axis_index("s") * nc + lax.axis_index("c")
        base = wid * b_per_w
        pltpu.sync_copy(idx_hbm.at[wid], idx_v)
        for j in range(chunks):
            pltpu.async_copy(table_hbm.at[idx_v.at[j]],
                             rows_v.at[pl.ds(j * 128, 128)], sem).wait()
        pltpu.sync_copy(rows_v, out_hbm.at[pl.ds(base, b_per_w)])

    return gather, nw, b_per_w, chunks


def kernel(z, W):
    zf = z.reshape(N, D)
    idx2, loss_arr = _tc_argmin(zf, W)
    gather, nw, b_per_w, chunks = _make_sc_gather()
    table = W.T
    idx3 = idx2.reshape(nw, chunks, 128)
    q = gather(table, idx3)
    quantized_st = q.reshape(B, H, W_SP, D).transpose(0, 3, 1, 2)
    loss = loss_arr[0, 0]
    return quantized_st, loss, loss


# speedup vs baseline: 1.8786x; 1.0384x over previous
import functools

import jax
import jax.numpy as jnp
from jax import lax
from jax.experimental import pallas as pl
from jax.experimental.pallas import tpu as pltpu
from jax.experimental.pallas import tpu_sc as plsc

B, H, W_SP, D, K = 8, 32, 32, 32, 8192
N = B * H * W_SP
TK = 1024
NUM_K = K // TK


def _argmin_body(z_ref, w_ref, idx_ref, loss_ref, zsq_ref, acc_v, acc_i):
    k = pl.program_id(0)
    zf = z_ref[...]

    @pl.when(k == 0)
    def _init():
        zsq_ref[...] = jnp.sum(zf * zf, axis=1, keepdims=True)
        acc_v[...] = jnp.full((N, 128), jnp.inf, jnp.float32)
        acc_i[...] = jnp.zeros((N, 128), jnp.int32)

    wblk = w_ref[...]
    csq = jnp.sum(wblk * wblk, axis=0, keepdims=True)
    s = lax.dot_general(zf, wblk, (((1,), (0,)), ((), ())),
                        preferred_element_type=jnp.float32)
    zsq = zsq_ref[...]
    lane = lax.broadcasted_iota(jnp.int32, (1, 128), 1)
    pairs = []
    for j in range(TK // 128):
        d = (zsq + csq[:, j * 128:(j + 1) * 128]) - 2.0 * s[:, j * 128:(j + 1) * 128]
        gidx = jnp.broadcast_to(lane + (k * TK + j * 128), (N, 128))
        pairs.append((d, gidx))
    while len(pairs) > 1:
        nxt = []
        for a in range(0, len(pairs), 2):
            (dv0, di0), (dv1, di1) = pairs[a], pairs[a + 1]
            better = dv1 < dv0
            nxt.append((jnp.minimum(dv0, dv1), jnp.where(better, di1, di0)))
        pairs = nxt
    dloc, iloc = pairs[0]
    better = dloc < acc_v[...]
    acc_v[...] = jnp.minimum(dloc, acc_v[...])
    acc_i[...] = jnp.where(better, iloc, acc_i[...])

    @pl.when(k == NUM_K - 1)
    def _finalize():
        av, ai = acc_v[...], acc_i[...]
        m = jnp.min(av, axis=1, keepdims=True)
        cand = jnp.where(av == m, ai, jnp.int32(2**30))
        idx_ref[...] = jnp.min(cand, axis=1, keepdims=True)
        loss_ref[...] = jnp.sum(m, keepdims=True).reshape(1, 1) / (N * D)


def _tc_argmin(zf, W):
    return pl.pallas_call(
        _argmin_body,
        grid=(NUM_K,),
        in_specs=[
            pl.BlockSpec((N, D), lambda k: (0, 0)),
            pl.BlockSpec((D, TK), lambda k: (0, k)),
        ],
        out_specs=[
            pl.BlockSpec((N, 1), lambda k: (0, 0)),
            pl.BlockSpec((1, 1), lambda k: (0, 0)),
        ],
        out_shape=[
            jax.ShapeDtypeStruct((N, 1), jnp.int32),
            jax.ShapeDtypeStruct((1, 1), jnp.float32),
        ],
        scratch_shapes=[
            pltpu.VMEM((N, 1), jnp.float32),
            pltpu.VMEM((N, 128), jnp.float32),
            pltpu.VMEM((N, 128), jnp.int32),
        ],
    )(zf, W)


def _make_sc_gather():
    info = plsc.get_sparse_core_info()
    nc, ns = info.num_cores, info.num_subcores
    nw = nc * ns
    b_per_w = N // nw
    chunks = b_per_w // 128
    mesh = plsc.VectorSubcoreMesh(core_axis_name="c", subcore_axis_name="s")

    @functools.partial(
        pl.kernel,
        mesh=mesh,
        compiler_params=pltpu.CompilerParams(use_tc_tiling_on_sc=False),
        out_type=jax.ShapeDtypeStruct((N, D), jnp.float32),
        scratch_types=[
            pltpu.VMEM((chunks, 128), jnp.int32),
            pltpu.VMEM((b_per_w, D), jnp.float32),
            pltpu.SemaphoreType.DMA,
        ],
    )
    def gather(table_hbm, idx_hbm, out_hbm, idx_v, rows_v, sem):
        wid = lax.axis_index("s") * nc + lax.axis_index("c")
        base = wid * b_per_w
        pltpu.sync_copy(idx_hbm.at[wid], idx_v)
        for j in range(chunks):
            pltpu.async_copy(table_hbm.at[idx_v.at[j]],
                             rows_v.at[pl.ds(j * 128, 128)], sem).wait()
        pltpu.sync_copy(rows_v, out_hbm.at[pl.ds(base, b_per_w)])

    return gather, nw, b_per_w, chunks


def kernel(z, W):
    zf = z.reshape(N, D)
    idx2, loss_arr = _tc_argmin(zf, W)
    gather, nw, b_per_w, chunks = _make_sc_gather()
    table = W.T
    idx3 = idx2.reshape(nw, chunks, 128)
    q = gather(table, idx3)
    quantized_st = q.reshape(B, H, W_SP, D).transpose(0, 3, 1, 2)
    loss = loss_arr[0, 0]
    return quantized_st, loss, loss


# speedup vs baseline: 1.9829x; 1.0555x over previous
import functools

import jax
import jax.numpy as jnp
from jax import lax
from jax.experimental import pallas as pl
from jax.experimental.pallas import tpu as pltpu
from jax.experimental.pallas import tpu_sc as plsc

B, H, W_SP, D, K = 8, 32, 32, 32, 8192
N = B * H * W_SP
TK = 1024
NUM_K = K // TK


def _argmin_body(z_ref, w_ref, idx_ref, loss_ref, zsq_ref, acc_v, acc_i):
    k = pl.program_id(0)
    zf = z_ref[...]

    @pl.when(k == 0)
    def _init():
        zsq_ref[...] = jnp.sum(zf * zf, axis=1, keepdims=True)
        acc_v[...] = jnp.full((N, 128), jnp.inf, jnp.float32)
        acc_i[...] = jnp.zeros((N, 128), jnp.int32)

    wblk = w_ref[...]
    csq = jnp.sum(wblk * wblk, axis=0, keepdims=True)
    s = lax.dot_general(zf, wblk, (((1,), (0,)), ((), ())),
                        preferred_element_type=jnp.float32)
    zsq = zsq_ref[...]
    lane = lax.broadcasted_iota(jnp.int32, (1, 128), 1)
    pairs = []
    for j in range(TK // 128):
        d = (zsq + csq[:, j * 128:(j + 1) * 128]) - 2.0 * s[:, j * 128:(j + 1) * 128]
        gidx = jnp.broadcast_to(lane + (k * TK + j * 128), (N, 128))
        pairs.append((d, gidx))
    while len(pairs) > 1:
        nxt = []
        for a in range(0, len(pairs), 2):
            (dv0, di0), (dv1, di1) = pairs[a], pairs[a + 1]
            better = dv1 < dv0
            nxt.append((jnp.minimum(dv0, dv1), jnp.where(better, di1, di0)))
        pairs = nxt
    dloc, iloc = pairs[0]
    better = dloc < acc_v[...]
    acc_v[...] = jnp.minimum(dloc, acc_v[...])
    acc_i[...] = jnp.where(better, iloc, acc_i[...])

    @pl.when(k == NUM_K - 1)
    def _finalize():
        av, ai = acc_v[...], acc_i[...]
        m = jnp.min(av, axis=1, keepdims=True)
        cand = jnp.where(av == m, ai, jnp.int32(2**30))
        idxv = jnp.min(cand, axis=1, keepdims=True)
        idx_ref[...] = idxv.reshape(N // 128, 128)
        loss_ref[...] = jnp.sum(m, keepdims=True).reshape(1, 1) / (N * D)


def _tc_argmin(zf, W):
    return pl.pallas_call(
        _argmin_body,
        grid=(NUM_K,),
        in_specs=[
            pl.BlockSpec((N, D), lambda k: (0, 0)),
            pl.BlockSpec((D, TK), lambda k: (0, k)),
        ],
        out_specs=[
            pl.BlockSpec((N // 128, 128), lambda k: (0, 0)),
            pl.BlockSpec((1, 1), lambda k: (0, 0)),
        ],
        out_shape=[
            jax.ShapeDtypeStruct((N // 128, 128), jnp.int32),
            jax.ShapeDtypeStruct((1, 1), jnp.float32),
        ],
        scratch_shapes=[
            pltpu.VMEM((N, 1), jnp.float32),
            pltpu.VMEM((N, 128), jnp.float32),
            pltpu.VMEM((N, 128), jnp.int32),
        ],
    )(zf, W)


def _make_sc_gather():
    info = plsc.get_sparse_core_info()
    nc, ns = info.num_cores, info.num_subcores
    nw = nc * ns
    b_per_w = N // nw
    chunks = b_per_w // 128
    mesh = plsc.VectorSubcoreMesh(core_axis_name="c", subcore_axis_name="s")

    @functools.partial(
        pl.kernel,
        mesh=mesh,
        compiler_params=pltpu.CompilerParams(use_tc_tiling_on_sc=False),
        out_type=jax.ShapeDtypeStruct((N, D), jnp.float32),
        scratch_types=[
            pltpu.VMEM((chunks, 128), jnp.int32),
            pltpu.VMEM((b_per_w, D), jnp.float32),
            pltpu.SemaphoreType.DMA,
        ],
    )
    def gather(table_hbm, idx_hbm, out_hbm, idx_v, rows_v, sem):
        wid = lax.axis_index("s") * nc + lax.axis_index("c")
        base = wid * b_per_w
        pltpu.sync_copy(idx_hbm.at[wid], idx_v)
        for j in range(chunks):
            pltpu.async_copy(table_hbm.at[idx_v.at[j]],
                             rows_v.at[pl.ds(j * 128, 128)], sem).wait()
        pltpu.sync_copy(rows_v, out_hbm.at[pl.ds(base, b_per_w)])

    return gather, nw, b_per_w, chunks


def kernel(z, W):
    zf = z.reshape(N, D)
    idx64, loss_arr = _tc_argmin(zf, W)
    gather, nw, b_per_w, chunks = _make_sc_gather()
    table = W.T
    idx3 = idx64.reshape(nw, chunks, 128)
    q = gather(table, idx3)
    quantized_st = q.reshape(B, H, W_SP, D).transpose(0, 3, 1, 2)
    loss = loss_arr[0, 0]
    return quantized_st, loss, loss


# speedup vs baseline: 2.1066x; 1.0624x over previous
import functools

import jax
import jax.numpy as jnp
from jax import lax
from jax.experimental import pallas as pl
from jax.experimental.pallas import tpu as pltpu
from jax.experimental.pallas import tpu_sc as plsc

B, H, W_SP, D, K = 8, 32, 32, 32, 8192
N = B * H * W_SP
TK = 1024
NUM_K = K // TK


def _argmin_body(z_ref, w_ref, idx_ref, loss_ref, zsq_ref, acc_v, acc_i):
    k = pl.program_id(0)
    zf = z_ref[...]

    @pl.when(k == 0)
    def _init():
        zsq_ref[...] = jnp.sum(zf * zf, axis=1, keepdims=True)
        acc_v[...] = jnp.full((N, 128), jnp.inf, jnp.float32)
        acc_i[...] = jnp.zeros((N, 128), jnp.int32)

    wblk = w_ref[...]
    csq = jnp.sum(wblk * wblk, axis=0, keepdims=True)
    s2 = lax.dot_general(zf, -2.0 * wblk, (((1,), (0,)), ((), ())),
                         preferred_element_type=jnp.float32)
    zsq = zsq_ref[...]
    lane = lax.broadcasted_iota(jnp.int32, (1, 128), 1)
    pairs = []
    for j in range(TK // 128):
        d = (zsq + csq[:, j * 128:(j + 1) * 128]) + s2[:, j * 128:(j + 1) * 128]
        gidx = jnp.broadcast_to(lane + (k * TK + j * 128), (N, 128))
        pairs.append((d, gidx))
    while len(pairs) > 1:
        nxt = []
        for a in range(0, len(pairs), 2):
            (dv0, di0), (dv1, di1) = pairs[a], pairs[a + 1]
            better = dv1 < dv0
            nxt.append((jnp.minimum(dv0, dv1), jnp.where(better, di1, di0)))
        pairs = nxt
    dloc, iloc = pairs[0]
    better = dloc < acc_v[...]
    acc_v[...] = jnp.minimum(dloc, acc_v[...])
    acc_i[...] = jnp.where(better, iloc, acc_i[...])

    @pl.when(k == NUM_K - 1)
    def _finalize():
        av, ai = acc_v[...], acc_i[...]
        m = jnp.min(av, axis=1, keepdims=True)
        cand = jnp.where(av == m, ai, jnp.int32(2**30))
        idxv = jnp.min(cand, axis=1, keepdims=True)
        idx_ref[...] = idxv.reshape(N // 128, 128)
        loss_ref[...] = jnp.sum(m, keepdims=True).reshape(1, 1) / (N * D)


def _tc_argmin(zf, W):
    return pl.pallas_call(
        _argmin_body,
        grid=(NUM_K,),
        in_specs=[
            pl.BlockSpec((N, D), lambda k: (0, 0)),
            pl.BlockSpec((D, TK), lambda k: (0, k)),
        ],
        out_specs=[
            pl.BlockSpec((N // 128, 128), lambda k: (0, 0)),
            pl.BlockSpec((1, 1), lambda k: (0, 0)),
        ],
        out_shape=[
            jax.ShapeDtypeStruct((N // 128, 128), jnp.int32),
            jax.ShapeDtypeStruct((1, 1), jnp.float32),
        ],
        scratch_shapes=[
            pltpu.VMEM((N, 1), jnp.float32),
            pltpu.VMEM((N, 128), jnp.float32),
            pltpu.VMEM((N, 128), jnp.int32),
        ],
    )(zf, W)


def _make_sc_gather():
    info = plsc.get_sparse_core_info()
    nc, ns = info.num_cores, info.num_subcores
    nw = nc * ns
    b_per_w = N // nw
    chunks = b_per_w // 128
    mesh = plsc.VectorSubcoreMesh(core_axis_name="c", subcore_axis_name="s")

    @functools.partial(
        pl.kernel,
        mesh=mesh,
        compiler_params=pltpu.CompilerParams(use_tc_tiling_on_sc=False),
        out_type=jax.ShapeDtypeStruct((N, D), jnp.float32),
        scratch_types=[
            pltpu.VMEM((chunks, 128), jnp.int32),
            pltpu.VMEM((b_per_w, D), jnp.float32),
            pltpu.SemaphoreType.DMA,
        ],
    )
    def gather(table_hbm, idx_hbm, out_hbm, idx_v, rows_v, sem):
        wid = lax.axis_index("s") * nc + lax.axis_index("c")
        base = wid * b_per_w
        pltpu.sync_copy(idx_hbm.at[wid], idx_v)
        for j in range(chunks):
            pltpu.async_copy(table_hbm.at[idx_v.at[j]],
                             rows_v.at[pl.ds(j * 128, 128)], sem).wait()
        pltpu.sync_copy(rows_v, out_hbm.at[pl.ds(base, b_per_w)])

    return gather, nw, b_per_w, chunks


def kernel(z, W):
    zf = z.reshape(N, D)
    idx64, loss_arr = _tc_argmin(zf, W)
    gather, nw, b_per_w, chunks = _make_sc_gather()
    table = W.T
    idx3 = idx64.reshape(nw, chunks, 128)
    q = gather(table, idx3)
    quantized_st = q.reshape(B, H, W_SP, D).transpose(0, 3, 1, 2)
    loss = loss_arr[0, 0]
    return quantized_st, loss, loss
